# baseline (speedup 1.0000x reference)
import functools

import jax
import jax.numpy as jnp
from jax import lax
from jax.experimental import pallas as pl
from jax.experimental.pallas import tpu as pltpu
from jax.experimental.pallas import tpu_sc as plsc

_N = 10000
_NP = 10240
_E = 320000
_C = 64
_FH = 32
_K = 10
_NS = 16
_ROWS = _NP // _NS
_SUB = 64
_CH = 128
_NCH = _E // _CH
_NCHM = 156
_NCHT = _NCHM + 1
_LANES = 16

_GDN = lax.GatherDimensionNumbers(
    offset_dims=(), collapsed_slice_dims=(0,), start_index_map=(0,))


def _mlp_body(a_s, x, W0, b0, W1, b1, W2, b2, W3, b3,
              g0, be0, g1, be1, g2, be2, o):
    av = a_s[0, 0]
    h = jnp.dot(x[...], W0[...], preferred_element_type=jnp.float32)
    h = h + b0[...][None, :]
    for W, b, g, be in ((W1, b1, g0, be0), (W2, b2, g1, be1), (W3, b3, g2, be2)):
        mu = jnp.mean(h, axis=0, keepdims=True)
        d = h - mu
        var = jnp.mean(d * d, axis=0, keepdims=True)
        hn = g[...][None, :] * d * lax.rsqrt(var + 1e-5) + be[...][None, :]
        hp = jnp.where(hn >= 0, hn, av * hn)
        h = jnp.dot(hp, W[...], preferred_element_type=jnp.float32)
        h = h + b[...][None, :]
    o[pl.ds(0, _N), :] = h[:, :_FH]
    o[pl.ds(_NP, _N), :] = h[:, _FH:]


def _mlp(x, W0, b0, W1, b1, W2, b2, W3, b3, g0, be0, g1, be1, g2, be2, a):
    a_s = jnp.reshape(a, (1, 1))
    specs = [pl.BlockSpec(memory_space=pltpu.SMEM)]
    specs += [pl.BlockSpec(memory_space=pltpu.VMEM)] * 15
    return pl.pallas_call(
        _mlp_body,
        out_shape=jax.ShapeDtypeStruct((2 * _NP, _FH), jnp.float32),
        in_specs=specs,
        out_specs=pl.BlockSpec(memory_space=pltpu.VMEM),
    )(a_s, x, W0, b0, W1, b1, W2, b2, W3, b3, g0, be0, g1, be1, g2, be2)


def _prop_body(h2f, src2d, dst2d, nrm2d, zrows, preds,
               nxt, rows0, rows1, rows2, rows3, sidx2, didx2, nrm2,
               semg0, semg1, semg2, semg3, sems0, sems1, sems2, sems3):
    s = lax.axis_index("s")
    c = lax.axis_index("c")
    row0 = s * _ROWS
    pbase = c * (_K + 1) * _NP
    cbase = s * _NCHM + jnp.minimum(s, 4)
    has_extra = s < 4

    pltpu.sync_copy(src2d.at[pl.ds(cbase, _NCHM)], sidx2.at[pl.ds(0, _NCHM)])
    pltpu.sync_copy(dst2d.at[pl.ds(cbase, _NCHM)], didx2.at[pl.ds(0, _NCHM)])
    pltpu.sync_copy(nrm2d.at[pl.ds(cbase, _NCHM)], nrm2.at[pl.ds(0, _NCHM)])

    @pl.when(has_extra)
    def _load_extra():
        pltpu.sync_copy(src2d.at[pl.ds(cbase + _NCHM, 1)],
                        sidx2.at[pl.ds(_NCHM, 1)])
        pltpu.sync_copy(dst2d.at[pl.ds(cbase + _NCHM, 1)],
                        didx2.at[pl.ds(_NCHM, 1)])
        pltpu.sync_copy(nrm2d.at[pl.ds(cbase + _NCHM, 1)],
                        nrm2.at[pl.ds(_NCHM, 1)])

    def _adj(ci, cc):
        for g in range(8):
            sl = pl.ds(g * _LANES, _LANES)
            sidx2[ci, sl] = sidx2[ci, sl] + pbase
        return cc
    lax.fori_loop(0, _NCHT, _adj, 0)

    for b in range(_ROWS // _SUB):
        pltpu.sync_copy(h2f.at[pl.ds(c * _NP + row0 + b * _SUB, _SUB)],
                        rows0.at[pl.ds(0, _SUB)])
        pltpu.sync_copy(rows0.at[pl.ds(0, _SUB)],
                        preds.at[pl.ds(pbase + row0 + b * _SUB, _SUB)])


    def _bcast(nv, le):
        bi = jnp.full((_LANES, 1), le, jnp.int32)
        return lax.gather(nv, bi, _GDN, (1,),
                          mode=lax.GatherScatterMode.PROMISE_IN_BOUNDS)

    def _mult(buf, ci):
        def _mg(g, cc):
            nv = nrm2[ci, pl.ds(g * _LANES, _LANES)]
            for le in range(_LANES):
                nb = _bcast(nv, le)
                e = g * _LANES + le
                for j in range(_FH // _LANES):
                    sl = pl.ds(j * _LANES, _LANES)
                    buf[e, sl] = buf[e, sl] * nb
            return cc
        lax.fori_loop(0, _CH // _LANES, _mg, 0)

    def _mult_static(buf, nrow):
        for g in range(_CH // _LANES):
            nv = nrow[pl.ds(g * _LANES, _LANES)]
            for le in range(_LANES):
                nb = _bcast(nv, le)
                e = g * _LANES + le
                for j in range(_FH // _LANES):
                    sl = pl.ds(j * _LANES, _LANES)
                    buf[e, sl] = buf[e, sl] * nb

    def _gstart(ci, buf, sem):
        return pltpu.async_copy(preds.at[sidx2.at[ci]], buf, sem)

    def _gwait(ci, buf, sem):
        pltpu.make_async_copy(preds.at[sidx2.at[ci]], buf, sem).wait()

    def _scat(ci, buf):
        pltpu.sync_copy(buf, nxt.at[didx2.at[ci]], add=True)

    def _sstart(ci, buf, sem):
        return pltpu.async_copy(buf, nxt.at[didx2.at[ci]], sem, add=True)

    def _swait(ci, buf, sem):
        pltpu.make_async_copy(buf, nxt.at[didx2.at[ci]], sem).wait()

    def _round(k, cc):
        pltpu.sync_copy(zrows.at[pl.ds(row0, _ROWS)],
                        nxt.at[pl.ds(row0, _ROWS)])
        plsc.subcore_barrier()

        bufs = (rows0, rows1, rows2, rows3)
        gsems = (semg0, semg1, semg2, semg3)
        ssems = (sems0, sems1, sems2, sems3)
        for b in range(4):
            _gstart(b, bufs[b], gsems[b])

        def _quad(qi, pc):
            ca = 4 * qi
            for b in range(4):
                _gwait(ca + b, bufs[b], gsems[b])
                _mult_static(bufs[b], nrm2.at[ca + b])
                _sstart(ca + b, bufs[b], ssems[b])
                if b >= 2:
                    bb = b - 2
                    _swait(ca + bb, bufs[bb], ssems[bb])

                    @pl.when(qi < _NCHM // 4 - 1)
                    def _look(bb=bb, ca=ca):
                        _gstart(ca + 4 + bb, bufs[bb], gsems[bb])
            for b in (2, 3):
                _swait(ca + b, bufs[b], ssems[b])

                @pl.when(qi < _NCHM // 4 - 1)
                def _look2(b=b, ca=ca):
                    _gstart(ca + 4 + b, bufs[b], gsems[b])
            return pc
        lax.fori_loop(0, _NCHM // 4, _quad, 0)

        @pl.when(has_extra)
        def _extra_chunk():
            _gstart(_NCHM, rows0, semg0).wait()
            _mult(rows0, _NCHM)
            _scat(_NCHM, rows0)
        plsc.subcore_barrier()

        out0 = pbase + k * _NP + row0
        pltpu.sync_copy(nxt.at[pl.ds(row0, _ROWS)],
                        preds.at[pl.ds(out0, _ROWS)])

        def _adv(ci, ac):
            for g in range(8):
                sl = pl.ds(g * _LANES, _LANES)
                sidx2[ci, sl] = sidx2[ci, sl] + _NP
            return ac
        lax.fori_loop(0, _NCHT, _adv, 0)
        return cc

    lax.fori_loop(1, _K + 1, _round, 0)


def _prop(h2f, src2d, dst2d, nrm2d):
    mesh = plsc.VectorSubcoreMesh(core_axis_name="c", subcore_axis_name="s")
    f = functools.partial(
        pl.kernel,
        out_type=jax.ShapeDtypeStruct((2 * (_K + 1) * _NP, _FH), jnp.float32),
        mesh=mesh,
        compiler_params=pltpu.CompilerParams(use_tc_tiling_on_sc=False),
        scratch_types=[
            pltpu.VMEM_SHARED((_NP, _FH), jnp.float32),
            pltpu.VMEM((_CH, _FH), jnp.float32),
            pltpu.VMEM((_CH, _FH), jnp.float32),
            pltpu.VMEM((_CH, _FH), jnp.float32),
            pltpu.VMEM((_CH, _FH), jnp.float32),
            pltpu.VMEM((_NCHT, _CH), jnp.int32),
            pltpu.VMEM((_NCHT, _CH), jnp.int32),
            pltpu.VMEM((_NCHT, _CH), jnp.float32),
        ] + [pltpu.SemaphoreType.DMA] * 8,
    )(_prop_body)
    zrows = jnp.zeros((_NP, _FH), jnp.float32)
    return f(h2f, src2d, dst2d, nrm2d, zrows)


def _score_body(bp_s, x0_ref, x1_ref, w0_ref, w1_ref, o):
    x0 = x0_ref[...]
    x1 = x1_ref[...]
    w0 = w0_ref[...][0]
    w1 = w1_ref[...][0]
    sc = (jnp.sum(x0 * w0[None, None, :], axis=-1)
          + jnp.sum(x1 * w1[None, None, :], axis=-1) + bp_s[0, 0])
    sc = jax.nn.sigmoid(sc)
    o0 = jnp.sum(sc[:, :, None] * x0, axis=0)
    o1 = jnp.sum(sc[:, :, None] * x1, axis=0)
    out = jnp.concatenate([o0, o1], axis=-1)
    m = jnp.max(out, axis=1, keepdims=True)
    z = out - m
    lse = jnp.log(jnp.sum(jnp.exp(z), axis=1, keepdims=True))
    o[...] = z - lse


def _score(pps0, pps1, Wp, bp):
    bn = 2000
    bp_s = jnp.reshape(bp, (1, 1))
    w0 = jnp.reshape(Wp[:_FH, 0], (1, _FH))
    w1 = jnp.reshape(Wp[_FH:, 0], (1, _FH))
    return pl.pallas_call(
        _score_body,
        grid=(_N // bn,),
        out_shape=jax.ShapeDtypeStruct((_N, _C), jnp.float32),
        in_specs=[
            pl.BlockSpec(memory_space=pltpu.SMEM),
            pl.BlockSpec((_K + 1, bn, _FH), lambda i: (0, i, 0)),
            pl.BlockSpec((_K + 1, bn, _FH), lambda i: (0, i, 0)),
            pl.BlockSpec((1, _FH), lambda i: (0, 0)),
            pl.BlockSpec((1, _FH), lambda i: (0, 0)),
        ],
        out_specs=pl.BlockSpec((bn, _C), lambda i: (i, 0)),
    )(bp_s, pps0, pps1, w0, w1)


def kernel(x, edge_index, norm, W0, b0, W1, b1, W2, b2, W3, b3,
           g0, be0, g1, be1, g2, be2, a, Wp, bp):
    h2f = _mlp(x, W0, b0, W1, b1, W2, b2, W3, b3,
               g0, be0, g1, be1, g2, be2, a)
    src2d = jnp.reshape(edge_index[0], (_NCH, _CH))
    dst2d = jnp.reshape(edge_index[1], (_NCH, _CH))
    nrm2d = jnp.reshape(norm, (_NCH, _CH))
    preds_flat = _prop(h2f, src2d, dst2d, nrm2d)
    pps = jnp.reshape(preds_flat, (2, _K + 1, _NP, _FH))[:, :, :_N, :]
    return _score(pps[0], pps[1], Wp, bp)


# speedup vs baseline: 1.0031x; 1.0031x over previous
import functools

import jax
import jax.numpy as jnp
from jax import lax
from jax.experimental import pallas as pl
from jax.experimental.pallas import tpu as pltpu
from jax.experimental.pallas import tpu_sc as plsc

_N = 10000
_NP = 10240
_E = 320000
_C = 64
_FH = 32
_K = 10
_NS = 16
_ROWS = _NP // _NS
_SUB = 64
_CH = 128
_NCH = _E // _CH
_NCHM = 156
_NCHT = _NCHM + 1
_LANES = 16

_GDN = lax.GatherDimensionNumbers(
    offset_dims=(), collapsed_slice_dims=(0,), start_index_map=(0,))


def _mlp_body(a_s, x, W0, b0, W1, b1, W2, b2, W3, b3,
              g0, be0, g1, be1, g2, be2, o):
    av = a_s[0, 0]
    h = jnp.dot(x[...], W0[...], preferred_element_type=jnp.float32)
    h = h + b0[...][None, :]
    for W, b, g, be in ((W1, b1, g0, be0), (W2, b2, g1, be1), (W3, b3, g2, be2)):
        mu = jnp.mean(h, axis=0, keepdims=True)
        d = h - mu
        var = jnp.mean(d * d, axis=0, keepdims=True)
        hn = g[...][None, :] * d * lax.rsqrt(var + 1e-5) + be[...][None, :]
        hp = jnp.where(hn >= 0, hn, av * hn)
        h = jnp.dot(hp, W[...], preferred_element_type=jnp.float32)
        h = h + b[...][None, :]
    o[pl.ds(0, _N), :] = h[:, :_FH]
    o[pl.ds(_NP, _N), :] = h[:, _FH:]


def _mlp(x, W0, b0, W1, b1, W2, b2, W3, b3, g0, be0, g1, be1, g2, be2, a):
    a_s = jnp.reshape(a, (1, 1))
    specs = [pl.BlockSpec(memory_space=pltpu.SMEM)]
    specs += [pl.BlockSpec(memory_space=pltpu.VMEM)] * 15
    return pl.pallas_call(
        _mlp_body,
        out_shape=jax.ShapeDtypeStruct((2 * _NP, _FH), jnp.float32),
        in_specs=specs,
        out_specs=pl.BlockSpec(memory_space=pltpu.VMEM),
    )(a_s, x, W0, b0, W1, b1, W2, b2, W3, b3, g0, be0, g1, be1, g2, be2)


def _prop_body(h2f, src2d, dst2d, nrm2d, zrows, preds,
               nxt, rows0, rows1, rows2, rows3, sidx2, didx2, nrm2,
               semg0, semg1, semg2, semg3, sems0, sems1, sems2, sems3):
    s = lax.axis_index("s")
    c = lax.axis_index("c")
    row0 = s * _ROWS
    pbase = c * (_K + 1) * _NP
    cbase = s * _NCHM + jnp.minimum(s, 4)
    has_extra = s < 4

    pltpu.sync_copy(src2d.at[pl.ds(cbase, _NCHM)], sidx2.at[pl.ds(0, _NCHM)])
    pltpu.sync_copy(dst2d.at[pl.ds(cbase, _NCHM)], didx2.at[pl.ds(0, _NCHM)])
    pltpu.sync_copy(nrm2d.at[pl.ds(cbase, _NCHM)], nrm2.at[pl.ds(0, _NCHM)])

    @pl.when(has_extra)
    def _load_extra():
        pltpu.sync_copy(src2d.at[pl.ds(cbase + _NCHM, 1)],
                        sidx2.at[pl.ds(_NCHM, 1)])
        pltpu.sync_copy(dst2d.at[pl.ds(cbase + _NCHM, 1)],
                        didx2.at[pl.ds(_NCHM, 1)])
        pltpu.sync_copy(nrm2d.at[pl.ds(cbase + _NCHM, 1)],
                        nrm2.at[pl.ds(_NCHM, 1)])

    def _adj(ci, cc):
        for g in range(8):
            sl = pl.ds(g * _LANES, _LANES)
            sidx2[ci, sl] = sidx2[ci, sl] + pbase
        return cc
    lax.fori_loop(0, _NCHT, _adj, 0)

    for b in range(_ROWS // _SUB):
        pltpu.sync_copy(h2f.at[pl.ds(c * _NP + row0 + b * _SUB, _SUB)],
                        rows0.at[pl.ds(0, _SUB)])
        pltpu.sync_copy(rows0.at[pl.ds(0, _SUB)],
                        preds.at[pl.ds(pbase + row0 + b * _SUB, _SUB)])


    def _bcast(nv, le):
        bi = jnp.full((_LANES, 1), le, jnp.int32)
        return lax.gather(nv, bi, _GDN, (1,),
                          mode=lax.GatherScatterMode.PROMISE_IN_BOUNDS)

    def _mult(buf, ci):
        def _mg(g, cc):
            nv = nrm2[ci, pl.ds(g * _LANES, _LANES)]
            for le in range(_LANES):
                nb = _bcast(nv, le)
                e = g * _LANES + le
                for j in range(_FH // _LANES):
                    sl = pl.ds(j * _LANES, _LANES)
                    buf[e, sl] = buf[e, sl] * nb
            return cc
        lax.fori_loop(0, _CH // _LANES, _mg, 0)

    def _mult_static(buf, nrow):
        for g in range(_CH // _LANES):
            nv = nrow[pl.ds(g * _LANES, _LANES)]
            for le in range(_LANES):
                nb = _bcast(nv, le)
                e = g * _LANES + le
                for j in range(_FH // _LANES):
                    sl = pl.ds(j * _LANES, _LANES)
                    buf[e, sl] = buf[e, sl] * nb

    def _gstart(ci, buf, sem):
        return pltpu.async_copy(preds.at[sidx2.at[ci]], buf, sem)

    def _gwait(ci, buf, sem):
        pltpu.make_async_copy(preds.at[sidx2.at[ci]], buf, sem).wait()

    def _scat(ci, buf):
        pltpu.sync_copy(buf, nxt.at[didx2.at[ci]], add=True)

    def _sstart(ci, buf, sem):
        return pltpu.async_copy(buf, nxt.at[didx2.at[ci]], sem, add=True)

    def _swait(ci, buf, sem):
        pltpu.make_async_copy(buf, nxt.at[didx2.at[ci]], sem).wait()

    def _round(k, cc):
        pltpu.sync_copy(zrows.at[pl.ds(row0, _ROWS)],
                        nxt.at[pl.ds(row0, _ROWS)])
        plsc.subcore_barrier()

        bufs = (rows0, rows1, rows2, rows3)
        gsems = (semg0, semg1, semg2, semg3)
        ssems = (sems0, sems1, sems2, sems3)
        for b in range(4):
            _gstart(b, bufs[b], gsems[b])

        def _quad(qi, pc):
            ca = 4 * qi
            for b in range(4):
                _gwait(ca + b, bufs[b], gsems[b])
                _mult_static(bufs[b], nrm2.at[ca + b])
                _sstart(ca + b, bufs[b], ssems[b])
                if b >= 2:
                    bb = b - 2
                    _swait(ca + bb, bufs[bb], ssems[bb])

                    @pl.when(qi < _NCHM // 4 - 1)
                    def _look(bb=bb, ca=ca):
                        _gstart(ca + 4 + bb, bufs[bb], gsems[bb])
            for b in (2, 3):
                _swait(ca + b, bufs[b], ssems[b])

                @pl.when(qi < _NCHM // 4 - 1)
                def _look2(b=b, ca=ca):
                    _gstart(ca + 4 + b, bufs[b], gsems[b])
            return pc
        lax.fori_loop(0, _NCHM // 4, _quad, 0)

        @pl.when(has_extra)
        def _extra_chunk():
            _gstart(_NCHM, rows0, semg0).wait()
            _mult(rows0, _NCHM)
            _scat(_NCHM, rows0)
        plsc.subcore_barrier()

        out0 = pbase + k * _NP + row0
        pltpu.sync_copy(nxt.at[pl.ds(row0, _ROWS)],
                        preds.at[pl.ds(out0, _ROWS)])

        def _adv(ci, ac):
            for g in range(8):
                sl = pl.ds(g * _LANES, _LANES)
                sidx2[ci, sl] = sidx2[ci, sl] + _NP
            return ac
        lax.fori_loop(0, _NCHT, _adv, 0)
        return cc

    lax.fori_loop(1, _K + 1, _round, 0)


def _prop(h2f, src2d, dst2d, nrm2d):
    mesh = plsc.VectorSubcoreMesh(core_axis_name="c", subcore_axis_name="s")
    f = functools.partial(
        pl.kernel,
        out_type=jax.ShapeDtypeStruct((2 * (_K + 1) * _NP, _FH), jnp.float32),
        mesh=mesh,
        compiler_params=pltpu.CompilerParams(use_tc_tiling_on_sc=False),
        scratch_types=[
            pltpu.VMEM_SHARED((_NP, _FH), jnp.float32),
            pltpu.VMEM((_CH, _FH), jnp.float32),
            pltpu.VMEM((_CH, _FH), jnp.float32),
            pltpu.VMEM((_CH, _FH), jnp.float32),
            pltpu.VMEM((_CH, _FH), jnp.float32),
            pltpu.VMEM((_NCHT, _CH), jnp.int32),
            pltpu.VMEM((_NCHT, _CH), jnp.int32),
            pltpu.VMEM((_NCHT, _CH), jnp.float32),
        ] + [pltpu.SemaphoreType.DMA] * 8,
    )(_prop_body)
    zrows = jnp.zeros((_NP, _FH), jnp.float32)
    return f(h2f, src2d, dst2d, nrm2d, zrows)


def _score_body(bp_s, x0_ref, x1_ref, w0_ref, w1_ref, o):
    x0 = x0_ref[...]
    x1 = x1_ref[...]
    w0 = w0_ref[...][0]
    w1 = w1_ref[...][0]
    sc = (jnp.sum(x0 * w0[None, None, :], axis=-1)
          + jnp.sum(x1 * w1[None, None, :], axis=-1) + bp_s[0, 0])
    sc = jax.nn.sigmoid(sc)
    o0 = jnp.sum(sc[:, :, None] * x0, axis=0)
    o1 = jnp.sum(sc[:, :, None] * x1, axis=0)
    out = jnp.concatenate([o0, o1], axis=-1)
    m = jnp.max(out, axis=1, keepdims=True)
    z = out - m
    lse = jnp.log(jnp.sum(jnp.exp(z), axis=1, keepdims=True))
    o[...] = z - lse


def _score(pps0, pps1, Wp, bp):
    bn = 1000
    bp_s = jnp.reshape(bp, (1, 1))
    w0 = jnp.reshape(Wp[:_FH, 0], (1, _FH))
    w1 = jnp.reshape(Wp[_FH:, 0], (1, _FH))
    return pl.pallas_call(
        _score_body,
        grid=(_N // bn,),
        out_shape=jax.ShapeDtypeStruct((_N, _C), jnp.float32),
        in_specs=[
            pl.BlockSpec(memory_space=pltpu.SMEM),
            pl.BlockSpec((_K + 1, bn, _FH), lambda i: (0, i, 0)),
            pl.BlockSpec((_K + 1, bn, _FH), lambda i: (0, i, 0)),
            pl.BlockSpec((1, _FH), lambda i: (0, 0)),
            pl.BlockSpec((1, _FH), lambda i: (0, 0)),
        ],
        out_specs=pl.BlockSpec((bn, _C), lambda i: (i, 0)),
    )(bp_s, pps0, pps1, w0, w1)


def kernel(x, edge_index, norm, W0, b0, W1, b1, W2, b2, W3, b3,
           g0, be0, g1, be1, g2, be2, a, Wp, bp):
    h2f = _mlp(x, W0, b0, W1, b1, W2, b2, W3, b3,
               g0, be0, g1, be1, g2, be2, a)
    src2d = jnp.reshape(edge_index[0], (_NCH, _CH))
    dst2d = jnp.reshape(edge_index[1], (_NCH, _CH))
    nrm2d = jnp.reshape(norm, (_NCH, _CH))
    preds_flat = _prop(h2f, src2d, dst2d, nrm2d)
    pps = jnp.reshape(preds_flat, (2, _K + 1, _NP, _FH))[:, :, :_N, :]
    return _score(pps[0], pps[1], Wp, bp)
